# baseline (speedup 1.0000x reference)
import functools

import jax
import jax.numpy as jnp
from jax import lax
from jax.experimental import pallas as pl
from jax.experimental.pallas import tpu as pltpu
from jax.experimental.pallas import tpu_sc as plsc

_F_BLOCK = 1024
_LANES = 16


def _coef_sc_kernel(S, K, E, se_ref, rw_ref, out_ref, se_v, rw_v, coef_v):
    cid = lax.axis_index("c")
    sid = lax.axis_index("s")

    @pl.when(jnp.logical_and(cid == 0, sid == 0))
    def _():
        pltpu.sync_copy(se_ref, se_v)
        pltpu.sync_copy(rw_ref, rw_v)
        nh = S // _LANES
        se_chunks = [se_v[pl.ds(i * _LANES, _LANES)] for i in range(K * nh)]
        rw_chunks = [rw_v[pl.ds(i * _LANES, _LANES)] for i in range(K * nh)]
        zero = jnp.zeros((_LANES,), jnp.float32)
        for e in range(E):
            for h in range(nh):
                acc = zero
                for k in range(K):
                    c = k * nh + h
                    acc = acc + jnp.where(se_chunks[c] == e, rw_chunks[c], 0.0)
                coef_v[pl.ds((e * S) + _LANES * h, _LANES)] = acc
        pltpu.sync_copy(coef_v, out_ref)


def _routing_coef(se_flat, rw_flat, E):
    S, K = se_flat.shape
    se_t = se_flat.T.reshape(-1)
    rw_t = rw_flat.T.reshape(-1)
    mesh = plsc.VectorSubcoreMesh(core_axis_name="c", subcore_axis_name="s")
    coef = pl.kernel(
        functools.partial(_coef_sc_kernel, S, K, E),
        mesh=mesh,
        out_type=jax.ShapeDtypeStruct((E * S,), jnp.float32),
        scratch_types=[
            pltpu.VMEM((S * K,), jnp.int32),
            pltpu.VMEM((S * K,), jnp.float32),
            pltpu.VMEM((E * S,), jnp.float32),
        ],
    )(se_t, rw_t)
    return coef.reshape(E, S)


def _ffn_kernel(x_ref, w1_ref, w2_ref, o_ref):
    fb = pl.program_id(1)

    h = jnp.dot(x_ref[:, :], w1_ref[0], preferred_element_type=jnp.float32)
    h = h * jax.nn.sigmoid(h)
    o = jnp.dot(h, w2_ref[0], preferred_element_type=jnp.float32)

    @pl.when(fb == 0)
    def _init():
        o_ref[0, :, :] = jnp.zeros_like(o_ref[0])

    o_ref[0, :, :] += o


def _combine_kernel(E, coef_ref, oall_ref, o_ref):
    acc = oall_ref[0] * coef_ref[0, :][:, None]
    for e in range(1, E):
        acc = acc + oall_ref[e] * coef_ref[e, :][:, None]
    o_ref[:, :] = acc


def kernel(x, routing_weights, selected_experts, w1, w2):
    shape = x.shape
    D = shape[-1]
    K = routing_weights.shape[-1]
    x_flat = x.reshape(-1, D)
    rw_flat = routing_weights.reshape(-1, K).astype(jnp.float32)
    se_flat = selected_experts.reshape(-1, K).astype(jnp.int32)
    S = x_flat.shape[0]
    E, _, F = w1.shape
    nf = F // _F_BLOCK

    coef = _routing_coef(se_flat, rw_flat, E)

    o_all = pl.pallas_call(
        _ffn_kernel,
        grid=(E, nf),
        in_specs=[
            pl.BlockSpec((S, D), lambda e, fb: (0, 0)),
            pl.BlockSpec((1, D, _F_BLOCK), lambda e, fb: (e, 0, fb)),
            pl.BlockSpec((1, _F_BLOCK, D), lambda e, fb: (e, fb, 0)),
        ],
        out_specs=pl.BlockSpec((1, S, D), lambda e, fb: (e, 0, 0)),
        out_shape=jax.ShapeDtypeStruct((E, S, D), jnp.float32),
    )(x_flat, w1, w2)

    out = pl.pallas_call(
        functools.partial(_combine_kernel, E),
        in_specs=[
            pl.BlockSpec((E, S), lambda: (0, 0)),
            pl.BlockSpec((E, S, D), lambda: (0, 0, 0)),
        ],
        out_specs=pl.BlockSpec((S, D), lambda: (0, 0)),
        out_shape=jax.ShapeDtypeStruct((S, D), jnp.float32),
    )(coef, o_all)

    return out.reshape(shape)


# speedup vs baseline: 1.3899x; 1.3899x over previous
import jax
import jax.numpy as jnp
from jax.experimental import pallas as pl

_F_BLOCK = 1024


def _ffn_kernel(se_ref, rw_ref, x_ref, w1_ref, w2_ref, o_ref):
    e = pl.program_id(0)
    fb = pl.program_id(1)

    h = jnp.dot(x_ref[:, :], w1_ref[0], preferred_element_type=jnp.float32)
    h = h * jax.nn.sigmoid(h)
    o = jnp.dot(h, w2_ref[0], preferred_element_type=jnp.float32)

    mask = (se_ref[:, :] == e).astype(jnp.float32)
    coef = jnp.sum(rw_ref[:, :] * mask, axis=1)
    contrib = o * coef[:, None]

    @pl.when(jnp.logical_and(e == 0, fb == 0))
    def _init():
        o_ref[:, :] = jnp.zeros_like(o_ref)

    o_ref[:, :] += contrib


def kernel(x, routing_weights, selected_experts, w1, w2):
    shape = x.shape
    D = shape[-1]
    K = routing_weights.shape[-1]
    x_flat = x.reshape(-1, D)
    rw_flat = routing_weights.reshape(-1, K).astype(jnp.float32)
    se_flat = selected_experts.reshape(-1, K).astype(jnp.int32)
    S = x_flat.shape[0]
    E, _, F = w1.shape
    nf = F // _F_BLOCK

    out = pl.pallas_call(
        _ffn_kernel,
        grid=(E, nf),
        in_specs=[
            pl.BlockSpec((S, K), lambda e, fb: (0, 0)),
            pl.BlockSpec((S, K), lambda e, fb: (0, 0)),
            pl.BlockSpec((S, D), lambda e, fb: (0, 0)),
            pl.BlockSpec((1, D, _F_BLOCK), lambda e, fb: (e, 0, fb)),
            pl.BlockSpec((1, _F_BLOCK, D), lambda e, fb: (e, fb, 0)),
        ],
        out_specs=pl.BlockSpec((S, D), lambda e, fb: (0, 0)),
        out_shape=jax.ShapeDtypeStruct((S, D), jnp.float32),
    )(se_flat, rw_flat, x_flat, w1, w2)

    return out.reshape(shape)
